# baseline (speedup 1.0000x reference)
import functools

import jax
import jax.numpy as jnp
from jax import lax
from jax.experimental import pallas as pl

B, C, H, Wg = 4, 512, 180, 180
N = H * Wg
HID = 128
NCLS = 10
NCLS_P = 16
P = 200
NT = 2048
NUM_T = (N + NT - 1) // NT


def _convhead_body(x_ref, ws_ref, bs_ref, whm_ref, bhm_ref, feat_ref, hm_ref):
    x = x_ref[0]
    feat = lax.dot_general(x, ws_ref[...], (((0,), (0,)), ((), ())),
                           preferred_element_type=jnp.float32)
    feat = jnp.maximum(feat + bs_ref[...], 0.0)
    feat_ref[0] = feat
    logits = lax.dot_general(whm_ref[...], feat, (((0,), (1,)), ((), ())),
                             preferred_element_type=jnp.float32)
    hm_ref[0] = jax.nn.sigmoid(logits + bhm_ref[...])


@jax.jit
def _convhead(x, W_shared, b_shared, W_hm_p, b_hm_p):
    return pl.pallas_call(
        _convhead_body,
        grid=(B, NUM_T),
        in_specs=[
            pl.BlockSpec((1, C, NT), lambda b, t: (b, 0, t)),
            pl.BlockSpec((C, HID), lambda b, t: (0, 0)),
            pl.BlockSpec((1, HID), lambda b, t: (0, 0)),
            pl.BlockSpec((HID, NCLS_P), lambda b, t: (0, 0)),
            pl.BlockSpec((NCLS_P, 1), lambda b, t: (0, 0)),
        ],
        out_specs=[
            pl.BlockSpec((1, NT, HID), lambda b, t: (b, t, 0)),
            pl.BlockSpec((1, NCLS_P, NT), lambda b, t: (b, 0, t)),
        ],
        out_shape=[
            jax.ShapeDtypeStruct((B, N, HID), jnp.float32),
            jax.ShapeDtypeStruct((B, NCLS_P, N), jnp.float32),
        ],
    )(x, W_shared, b_shared, W_hm_p, b_hm_p)


def kernel(bev_feat, W_shared, b_shared, W_hm, b_hm, W_center, b_center,
           W_height, b_height, W_dim, b_dim, W_rot, b_rot, W_qhm, b_qhm,
           num_proposals):
    x = bev_feat.reshape(B, C, N)
    W_hm_p = jnp.concatenate(
        [W_hm, jnp.zeros((HID, NCLS_P - NCLS), jnp.float32)], axis=1)
    b_hm_p = jnp.concatenate(
        [b_hm, jnp.full((NCLS_P - NCLS,), jnp.finfo(jnp.float32).min)],
        axis=0).reshape(NCLS_P, 1)
    feat, hm = _convhead(x, W_shared, b_shared.reshape(1, HID), W_hm_p, b_hm_p)

    scores_flat = hm.reshape(B, NCLS_P * N)
    top_scores, top_idx = lax.top_k(scores_flat, P)
    top_idx = top_idx + (num_proposals - P)
    labels = top_idx // N
    pos = top_idx % N
    q = jnp.take_along_axis(feat, pos[:, :, None], axis=1)
    W_cat = jnp.concatenate([W_center, W_height, W_dim, W_rot, W_qhm], axis=1)
    b_cat = jnp.concatenate([b_center, b_height, b_dim, b_rot, b_qhm], axis=0)
    heads = jnp.einsum('bpk,ko->bop', q, W_cat) + b_cat[None, :, None]
    preds = jnp.concatenate([top_scores[:, None, :], heads], axis=1)
    return (preds, labels, pos)
